# baseline (speedup 1.0000x reference)
import functools
import math

import jax
import jax.numpy as jnp
from jax import lax
from jax.experimental import pallas as pl
from jax.experimental.pallas import tpu as pltpu
from jax.experimental.pallas import tpu_sc as plsc

EMB = 64
HID = 64
TBL_W = 128

NC = 2
NS = 16
NW = NC * NS

CHUNK = 32
KBUF = 5
BLOCK = CHUNK * KBUF
NBUF = 4


def _table_transform(embT, w2p):
    v = embT.shape[1]
    bm = 32768

    def body(e_ref, w_ref, o_ref):
        o_ref[...] = lax.dot_general(
            e_ref[...], w_ref[...], (((0,), (0,)), ((), ())),
            preferred_element_type=jnp.float32,
        )

    return pl.pallas_call(
        body,
        grid=(pl.cdiv(v, bm),),
        in_specs=[
            pl.BlockSpec((EMB, bm), lambda i: (0, i)),
            pl.BlockSpec((EMB, TBL_W), lambda i: (0, 0)),
        ],
        out_specs=pl.BlockSpec((bm, TBL_W), lambda i: (i, 0)),
        out_shape=jax.ShapeDtypeStruct((v, TBL_W), jnp.float32),
    )(embT, w2p)


@functools.lru_cache(maxsize=None)
def _gather_call(n, v, b_dim, l_dim):
    per_w = n // NW
    nblk = per_w // BLOCK
    assert per_w % BLOCK == 0 and nblk % NBUF == 0 and nblk >= 2 * NBUF
    assert b_dim == 4096

    mesh = plsc.VectorSubcoreMesh(core_axis_name="c", subcore_axis_name="s")

    @functools.partial(
        pl.kernel,
        mesh=mesh,
        out_type=jax.ShapeDtypeStruct((n, TBL_W), jnp.float32),
        scratch_types=[
            pltpu.VMEM((3 * 4096,), jnp.int32),
            pltpu.VMEM((NBUF, BLOCK, TBL_W), jnp.float32),
        ] + [pltpu.SemaphoreType.DMA] * (2 * NBUF),
    )
    def gather(st_hbm, table_hbm, out_hbm, stw_v, rows_v, *sems):
        gsems, ssems = sems[:NBUF], sems[NBUF:]
        wid = lax.axis_index("s") * NC + lax.axis_index("c")
        base = wid * per_w

        f0 = wid * per_w
        l0 = jnp.minimum(f0 >> 12, l_dim - 3)
        for r in range(3):
            pltpu.sync_copy(st_hbm.at[l0 + r],
                            stw_v.at[pl.ds(r * 4096, 4096)])
        off = f0 - l0 * 4096

        def fire(b, q):
            for j in range(KBUF):
                pltpu.make_async_copy(
                    table_hbm.at[
                        stw_v.at[pl.ds(off + (b * KBUF + j) * CHUNK, CHUNK)]],
                    rows_v.at[q, pl.ds(j * CHUNK, CHUNK)],
                    gsems[q],
                ).start()

        def wait_gathers(b, q):
            for j in range(KBUF):
                pltpu.make_async_copy(
                    table_hbm.at[
                        stw_v.at[pl.ds(off + (b * KBUF + j) * CHUNK, CHUNK)]],
                    rows_v.at[q, pl.ds(j * CHUNK, CHUNK)],
                    gsems[q],
                ).wait()

        def store(b, q):
            pltpu.make_async_copy(
                rows_v.at[q],
                out_hbm.at[pl.ds(base + b * BLOCK, BLOCK)],
                ssems[q],
            ).start()

        def wait_store(b, q):
            pltpu.make_async_copy(
                rows_v.at[q],
                out_hbm.at[pl.ds(base + b * BLOCK, BLOCK)],
                ssems[q],
            ).wait()

        fire(0, 0)
        fire(1, 1)
        wait_gathers(0, 0)
        store(0, 0)
        fire(2, 2)
        wait_gathers(1, 1)
        store(1, 1)
        fire(3, 3)

        def body(i2, carry):
            for q in range(NBUF):
                b = NBUF * i2 + q
                qs = (q - 2) % NBUF
                wait_gathers(b - 2, qs)
                store(b - 2, qs)
                wait_store(b - NBUF, q)
                fire(b, q)
            return carry

        lax.fori_loop(1, nblk // NBUF, body, 0)

        wait_gathers(nblk - 2, (nblk - 2) % NBUF)
        store(nblk - 2, (nblk - 2) % NBUF)
        wait_gathers(nblk - 1, (nblk - 1) % NBUF)
        store(nblk - 1, (nblk - 1) % NBUF)
        for b in range(nblk - NBUF, nblk):
            wait_store(b, b % NBUF)

    return gather


def _proj_out(x2, posw, l_off, b_dim, l_tot, prev=None):
    bc = 4096
    l_dim = x2.shape[0] // b_dim
    grid = (l_dim, b_dim // bc)

    def body(*refs):
        x_ref, p_ref, eye_ref = refs[0], refs[1], refs[2]
        o_ref = refs[-1]
        l = pl.program_id(0)
        y = x_ref[...][:, :EMB] + p_ref[pl.ds(l_off + l, 1), :]
        o_ref[0] = lax.dot_general(
            eye_ref[...], y, (((1,), (1,)), ((), ())),
            preferred_element_type=jnp.float32)

    in_specs = [
        pl.BlockSpec((bc, TBL_W), lambda l, j: (l * (b_dim // bc) + j, 0)),
        pl.BlockSpec((l_tot, EMB), lambda l, j: (0, 0)),
        pl.BlockSpec((HID, HID), lambda l, j: (0, 0)),
    ]
    args = [x2, posw, jnp.eye(HID, dtype=jnp.float32)]
    aliases = {}
    if prev is not None:
        in_specs.append(pl.BlockSpec(memory_space=pl.ANY))
        args.append(prev)
        aliases = {3: 0}

    return pl.pallas_call(
        body,
        grid=grid,
        in_specs=in_specs,
        out_specs=pl.BlockSpec((1, HID, bc), lambda l, j: (l_off + l, 0, j)),
        out_shape=jax.ShapeDtypeStruct((l_tot, HID, b_dim), jnp.float32),
        input_output_aliases=aliases,
    )(*args)


def kernel(sent1, sent2, emb, pos_emb, W):
    b, l1 = sent1.shape
    l2 = sent2.shape[1]
    scale = math.sqrt(emb.shape[1])

    w2p = jnp.concatenate(
        [scale * W.T, jnp.zeros((EMB, TBL_W - HID), jnp.float32)], axis=1)
    table2p = _table_transform(emb.T, w2p)

    n = b * l1
    hl = l1 // 2
    gc_h = _gather_call(b * hl, table2p.shape[0], b, hl)
    gc_f = _gather_call(n, table2p.shape[0], b, l2)
    st1 = sent1.T
    x1a = gc_h(st1[:hl], table2p)
    x1b = gc_h(st1[hl:], table2p)
    x2 = gc_f(sent2.T, table2p)

    posw = pos_emb[:l1] @ W.T
    o1a = _proj_out(x1a, posw, 0, b, l1)
    o1p = _proj_out(x1b, posw, hl, b, l1, prev=o1a)
    o2p = _proj_out(x2, posw, 0, b, l2)
    o1 = jnp.transpose(o1p, (2, 0, 1))
    o2 = jnp.transpose(o2p, (2, 0, 1))
    return (o1, o2)


# speedup vs baseline: 1.0116x; 1.0116x over previous
import functools
import math

import jax
import jax.numpy as jnp
from jax import lax
from jax.experimental import pallas as pl
from jax.experimental.pallas import tpu as pltpu
from jax.experimental.pallas import tpu_sc as plsc

EMB = 64
HID = 64
TBL_W = 128

NC = 2
NS = 16
NW = NC * NS

CHUNK = 32
KBUF = 5
BLOCK = CHUNK * KBUF
NBUF = 4


def _table_transform(embT, w2p):
    v = embT.shape[1]
    bm = 32768

    def body(e_ref, w_ref, o_ref):
        o_ref[...] = lax.dot_general(
            e_ref[...], w_ref[...], (((0,), (0,)), ((), ())),
            preferred_element_type=jnp.float32,
        )

    return pl.pallas_call(
        body,
        grid=(pl.cdiv(v, bm),),
        in_specs=[
            pl.BlockSpec((EMB, bm), lambda i: (0, i)),
            pl.BlockSpec((EMB, TBL_W), lambda i: (0, 0)),
        ],
        out_specs=pl.BlockSpec((bm, TBL_W), lambda i: (i, 0)),
        out_shape=jax.ShapeDtypeStruct((v, TBL_W), jnp.float32),
    )(embT, w2p)


@functools.lru_cache(maxsize=None)
def _gather_call(n, v, b_dim, l_dim):
    per_w = n // NW
    nblk = per_w // BLOCK
    assert per_w % BLOCK == 0 and nblk % NBUF == 0 and nblk >= 2 * NBUF
    assert b_dim == 4096

    mesh = plsc.VectorSubcoreMesh(core_axis_name="c", subcore_axis_name="s")

    @functools.partial(
        pl.kernel,
        mesh=mesh,
        out_type=jax.ShapeDtypeStruct((n, TBL_W), jnp.float32),
        scratch_types=[
            pltpu.VMEM((3 * 4096,), jnp.int32),
            pltpu.VMEM((NBUF, BLOCK, TBL_W), jnp.float32),
        ] + [pltpu.SemaphoreType.DMA] * (2 * NBUF),
    )
    def gather(st_hbm, table_hbm, out_hbm, stw_v, rows_v, *sems):
        gsems, ssems = sems[:NBUF], sems[NBUF:]
        wid = lax.axis_index("s") * NC + lax.axis_index("c")
        base = wid * per_w

        f0 = wid * per_w
        l0 = jnp.minimum(f0 >> 12, l_dim - 3)
        for r in range(3):
            pltpu.sync_copy(st_hbm.at[l0 + r],
                            stw_v.at[pl.ds(r * 4096, 4096)])
        off = f0 - l0 * 4096

        def fire(b, q):
            for j in range(KBUF):
                pltpu.make_async_copy(
                    table_hbm.at[
                        stw_v.at[pl.ds(off + (b * KBUF + j) * CHUNK, CHUNK)]],
                    rows_v.at[q, pl.ds(j * CHUNK, CHUNK)],
                    gsems[q],
                ).start()

        def wait_gathers(b, q):
            for j in range(KBUF):
                pltpu.make_async_copy(
                    table_hbm.at[
                        stw_v.at[pl.ds(off + (b * KBUF + j) * CHUNK, CHUNK)]],
                    rows_v.at[q, pl.ds(j * CHUNK, CHUNK)],
                    gsems[q],
                ).wait()

        def store(b, q):
            pltpu.make_async_copy(
                rows_v.at[q],
                out_hbm.at[pl.ds(base + b * BLOCK, BLOCK)],
                ssems[q],
            ).start()

        def wait_store(b, q):
            pltpu.make_async_copy(
                rows_v.at[q],
                out_hbm.at[pl.ds(base + b * BLOCK, BLOCK)],
                ssems[q],
            ).wait()

        fire(0, 0)
        fire(1, 1)
        wait_gathers(0, 0)
        store(0, 0)
        fire(2, 2)
        wait_gathers(1, 1)
        store(1, 1)
        fire(3, 3)

        def body(i2, carry):
            for q in range(NBUF):
                b = NBUF * i2 + q
                qs = (q - 2) % NBUF
                wait_gathers(b - 2, qs)
                store(b - 2, qs)
                wait_store(b - NBUF, q)
                fire(b, q)
            return carry

        lax.fori_loop(1, nblk // NBUF, body, 0)

        wait_gathers(nblk - 2, (nblk - 2) % NBUF)
        store(nblk - 2, (nblk - 2) % NBUF)
        wait_gathers(nblk - 1, (nblk - 1) % NBUF)
        store(nblk - 1, (nblk - 1) % NBUF)
        for b in range(nblk - NBUF, nblk):
            wait_store(b, b % NBUF)

    return gather


def _proj_out(x2, posw, l_off, b_dim, l_dim):
    bc = 4096
    grid = (l_dim, b_dim // bc)

    def body(x_ref, p_ref, eye_ref, o_ref):
        l = pl.program_id(0)
        y = x_ref[...][:, :EMB] + p_ref[pl.ds(l, 1), :]
        o_ref[0] = lax.dot_general(
            eye_ref[...], y, (((1,), (1,)), ((), ())),
            preferred_element_type=jnp.float32)

    return pl.pallas_call(
        body,
        grid=grid,
        in_specs=[
            pl.BlockSpec((bc, TBL_W),
                         lambda l, j: (l_off * (b_dim // bc) + l * (b_dim // bc) + j, 0)),
            pl.BlockSpec((l_dim, EMB), lambda l, j: (0, 0)),
            pl.BlockSpec((HID, HID), lambda l, j: (0, 0)),
        ],
        out_specs=pl.BlockSpec((1, HID, bc), lambda l, j: (l, 0, j)),
        out_shape=jax.ShapeDtypeStruct((l_dim, HID, b_dim), jnp.float32),
    )(x2, posw, jnp.eye(HID, dtype=jnp.float32))


def kernel(sent1, sent2, emb, pos_emb, W):
    b, l1 = sent1.shape
    l2 = sent2.shape[1]
    scale = math.sqrt(emb.shape[1])

    w2p = jnp.concatenate(
        [scale * W.T, jnp.zeros((EMB, TBL_W - HID), jnp.float32)], axis=1)
    table2p = _table_transform(emb.T, w2p)

    n = b * l1
    gcall = _gather_call(n, table2p.shape[0], b, l1)
    x1 = gcall(sent1.T, table2p)
    x2 = gcall(sent2.T, table2p)

    posw = pos_emb[:l1] @ W.T
    o1p = _proj_out(x1, posw, 0, b, l1)
    o2p = _proj_out(x2, posw, 0, b, l2)
    o1 = jnp.transpose(o1p, (2, 0, 1))
    o2 = jnp.transpose(o2p, (2, 0, 1))
    return (o1, o2)
